# baseline (speedup 1.0000x reference)
import functools

import jax
import jax.numpy as jnp
from jax import lax
from jax.experimental import pallas as pl
from jax.experimental.pallas import tpu as pltpu
from jax.experimental.pallas import tpu_sc as plsc


def _gather_call(B, D, b_per_w, NC, NS):
    mesh = plsc.VectorSubcoreMesh(core_axis_name="c", subcore_axis_name="s")
    half = b_per_w // 2

    @functools.partial(
        pl.kernel,
        mesh=mesh,
        out_type=jax.ShapeDtypeStruct((B, D), jnp.float32),
        scratch_types=[
            pltpu.VMEM((b_per_w,), jnp.int32),
            pltpu.VMEM((half, D), jnp.float32),
            pltpu.VMEM_SHARED((NS * half, D), jnp.float32),
            pltpu.SemaphoreType.DMA,
            pltpu.SemaphoreType.DMA,
        ],
    )
    def k(table_hbm, idx_hbm, out_hbm, idx_v, rows_v, shared_v, sem_a, sem_b):
        cid = lax.axis_index("c")
        sid = lax.axis_index("s")
        wid = sid * NC + cid
        base = wid * b_per_w
        sbase = sid * half
        pltpu.sync_copy(idx_hbm.at[pl.ds(base, b_per_w)], idx_v)

        def fire(g, carry):
            vec = idx_v[pl.ds(g * 16, 16)]
            for j in range(0, 16, 2):
                row_a = vec[j]
                row_b = vec[j + 1]
                i = g * 8 + j // 2
                pltpu.async_copy(
                    table_hbm.at[pl.ds(row_a, 1)],
                    rows_v.at[pl.ds(i, 1)],
                    sem_a,
                )
                pltpu.async_copy(
                    table_hbm.at[pl.ds(row_b, 1)],
                    shared_v.at[pl.ds(sbase + i, 1)],
                    sem_b,
                )
            return carry

        lax.fori_loop(0, b_per_w // 16, fire, 0)
        pltpu.make_async_copy(
            table_hbm.at[pl.ds(0, half)], rows_v, sem_a
        ).wait()
        pltpu.make_async_copy(
            table_hbm.at[pl.ds(0, half)],
            shared_v.at[pl.ds(sbase, half)],
            sem_b,
        ).wait()
        pltpu.sync_copy(rows_v, out_hbm.at[pl.ds(base, half)])
        pltpu.sync_copy(
            shared_v.at[pl.ds(sbase, half)],
            out_hbm.at[pl.ds(base + half, half)],
        )

    return k


def kernel(lookup, table):
    B, = lookup.shape
    V, D = table.shape
    info = plsc.get_sparse_core_info()
    NC, NS = info.num_cores, info.num_subcores
    NW = NC * NS
    b_per_w = B // NW
    idx = lookup.astype(jnp.int32)
    l = jnp.arange(b_per_w, dtype=jnp.int32)
    p = (l % 2) * (b_per_w // 2) + (l // 16) * 8 + (l % 16) // 2
    P = (jnp.arange(NW, dtype=jnp.int32)[:, None] * b_per_w + p[None, :]).reshape(B)
    deint = jnp.take(idx, P, axis=0)
    return _gather_call(B, D, b_per_w, NC, NS)(table, deint)


# speedup vs baseline: 1.0492x; 1.0492x over previous
import functools

import jax
import jax.numpy as jnp
from jax import lax
from jax.experimental import pallas as pl
from jax.experimental.pallas import tpu as pltpu
from jax.experimental.pallas import tpu_sc as plsc

NSEM = 8


def _gather_call(B, D, b_per_w, NC):
    mesh = plsc.VectorSubcoreMesh(core_axis_name="c", subcore_axis_name="s")

    @functools.partial(
        pl.kernel,
        mesh=mesh,
        out_type=jax.ShapeDtypeStruct((B, D), jnp.float32),
        scratch_types=[
            pltpu.VMEM((b_per_w,), jnp.int32),
            pltpu.VMEM((b_per_w, D), jnp.float32),
        ]
        + [pltpu.SemaphoreType.DMA] * NSEM,
    )
    def k(table_hbm, idx_hbm, out_hbm, idx_v, rows_v, *sems):
        wid = lax.axis_index("s") * NC + lax.axis_index("c")
        base = wid * b_per_w
        pltpu.sync_copy(idx_hbm.at[pl.ds(base, b_per_w)], idx_v)

        def fire(g, carry):
            vec = idx_v[pl.ds(g * 16, 16)]
            for j in range(16):
                row = vec[j]
                pltpu.async_copy(
                    table_hbm.at[pl.ds(row, 1)],
                    rows_v.at[pl.ds(g * 16 + j, 1)],
                    sems[j % NSEM],
                )
            return carry

        lax.fori_loop(0, b_per_w // 16, fire, 0)
        per_sem = b_per_w // NSEM
        for j in range(NSEM):
            pltpu.make_async_copy(
                table_hbm.at[pl.ds(0, per_sem)],
                rows_v.at[pl.ds(0, per_sem)],
                sems[j],
            ).wait()
        pltpu.sync_copy(rows_v, out_hbm.at[pl.ds(base, b_per_w)])

    return k


def kernel(lookup, table):
    B, = lookup.shape
    V, D = table.shape
    info = plsc.get_sparse_core_info()
    NW = info.num_cores * info.num_subcores
    b_per_w = B // NW
    idx = lookup.astype(jnp.int32)
    return _gather_call(B, D, b_per_w, info.num_cores)(table, idx)
